# baseline (speedup 1.0000x reference)
import functools

import jax
import jax.numpy as jnp
from jax import lax
from jax.experimental import pallas as pl
from jax.experimental.pallas import tpu as pltpu
from jax.experimental.pallas import tpu_sc as plsc

EPS = 1e-5
TM = 512

_SC_CORES = 2
_SC_SUBCORES = 16
_SC_WORKERS = _SC_CORES * _SC_SUBCORES


def _dist_argmin_body(x_ref, mean_ref, denom_ref, gamma_ref, beta_ref,
                      cb2_ref, b2_ref, a2_ref, idx_ref):
    xn = (x_ref[...] - mean_ref[...]) / denom_ref[...] * gamma_ref[...] + beta_ref[...]
    s2 = lax.dot_general(xn, cb2_ref[...], (((1,), (1,)), ((), ())),
                         preferred_element_type=jnp.float32)
    d2 = a2_ref[...] + b2_ref[...] - s2
    m2 = jnp.min(d2, axis=1, keepdims=True)
    s = jnp.sqrt(m2)
    sb = lax.bitcast_convert_type(s, jnp.int32)
    h = (lax.bitcast_convert_type(sb + 1, jnp.float32) - s) * 0.5
    c = s * 4097.0
    s_hi = c - (c - s)
    s_lo = s - s_hi
    p = s * s
    e = ((s_hi * s_hi - p) + 2.0 * (s_hi * s_lo)) + s_lo * s_lo
    r = 2.0 * (s * h) + (h * h + e)
    near = (d2 - p) <= r
    iota = lax.broadcasted_iota(jnp.int32, d2.shape, 1)
    idx_ref[...] = jnp.min(jnp.where(near, iota, d2.shape[1]), axis=1)


def _nearest_indices(xn_inputs, n_tokens, dim, num_codes):
    x, mean, denom, gamma2, beta2, codebook, b2, a2 = xn_inputs
    return pl.pallas_call(
        _dist_argmin_body,
        grid=(n_tokens // TM,),
        in_specs=[
            pl.BlockSpec((TM, dim), lambda i: (i, 0)),
            pl.BlockSpec((1, dim), lambda i: (0, 0)),
            pl.BlockSpec((1, dim), lambda i: (0, 0)),
            pl.BlockSpec((1, dim), lambda i: (0, 0)),
            pl.BlockSpec((1, dim), lambda i: (0, 0)),
            pl.BlockSpec((num_codes, dim), lambda i: (0, 0)),
            pl.BlockSpec((1, num_codes), lambda i: (0, 0)),
            pl.BlockSpec((TM, 1), lambda i: (i, 0)),
        ],
        out_specs=pl.BlockSpec((TM,), lambda i: (i,)),
        out_shape=jax.ShapeDtypeStruct((n_tokens,), jnp.int32),
    )(x, mean, denom, gamma2, beta2, codebook, b2, a2)


@functools.lru_cache(maxsize=None)
def _make_sc_gather(num_codes, dim, n_tokens):
    b_per_w = n_tokens // _SC_WORKERS
    mesh = plsc.VectorSubcoreMesh(core_axis_name="c", subcore_axis_name="s")

    @functools.partial(
        pl.kernel, mesh=mesh,
        out_type=jax.ShapeDtypeStruct((n_tokens, dim), jnp.float32),
        scratch_types=[
            pltpu.VMEM((b_per_w,), jnp.int32),
            pltpu.VMEM((b_per_w, dim), jnp.float32),
            pltpu.SemaphoreType.DMA,
        ],
    )
    def gather(table_hbm, idx_hbm, out_hbm, idx_v, rows_v, sem):
        wid = lax.axis_index("s") * _SC_CORES + lax.axis_index("c")
        base = wid * b_per_w
        pltpu.sync_copy(idx_hbm.at[pl.ds(base, b_per_w)], idx_v)
        pltpu.async_copy(table_hbm.at[idx_v], rows_v, sem).wait()
        pltpu.sync_copy(rows_v, out_hbm.at[pl.ds(base, b_per_w)])

    return gather


def kernel(x, bn_gamma, bn_beta, codebook):
    n_tokens, dim = x.shape
    num_codes = codebook.shape[0]
    mean = jnp.mean(x, axis=0, keepdims=True)
    var = jnp.mean((x - mean) ** 2, axis=0, keepdims=True)
    denom = jnp.sqrt(var + EPS)
    b2 = jnp.sum(codebook * codebook, axis=-1)[None, :]
    xn_stat = (x - mean) / denom * bn_gamma + bn_beta
    a2 = jnp.sum(xn_stat * xn_stat, axis=-1, keepdims=True)
    indices = _nearest_indices(
        (x, mean, denom, bn_gamma[None, :], bn_beta[None, :],
         codebook + codebook, b2, a2),
        n_tokens, dim, num_codes)
    codes = _make_sc_gather(num_codes, dim, n_tokens)(codebook, indices)
    return codes, indices.reshape(n_tokens, 1)


# speedup vs baseline: 1.7616x; 1.7616x over previous
import functools

import jax
import jax.numpy as jnp
from jax import lax
from jax.experimental import pallas as pl
from jax.experimental.pallas import tpu as pltpu
from jax.experimental.pallas import tpu_sc as plsc

EPS = 1e-5
TM = 512

_SC_CORES = 2
_SC_SUBCORES = 16
_SC_WORKERS = _SC_CORES * _SC_SUBCORES


def _dist_argmin_body(x_ref, mean_ref, denom_ref, gamma_ref, beta_ref,
                      cb2_ref, b2_ref, a2_ref, idx_ref):
    xn = (x_ref[...] - mean_ref[...]) / denom_ref[...] * gamma_ref[...] + beta_ref[...]
    s2 = lax.dot_general(xn, cb2_ref[...], (((1,), (1,)), ((), ())),
                         preferred_element_type=jnp.float32)
    d2 = a2_ref[...] + b2_ref[...] - s2
    dist = jnp.sqrt(d2)
    m = jnp.min(dist, axis=1, keepdims=True)
    iota = lax.broadcasted_iota(jnp.int32, dist.shape, 1)
    idx_ref[...] = jnp.min(jnp.where(dist == m, iota, dist.shape[1]), axis=1)


def _nearest_indices(xn_inputs, n_tokens, dim, num_codes):
    x, mean, denom, gamma2, beta2, codebook, b2, a2 = xn_inputs
    return pl.pallas_call(
        _dist_argmin_body,
        grid=(n_tokens // TM,),
        in_specs=[
            pl.BlockSpec((TM, dim), lambda i: (i, 0)),
            pl.BlockSpec((1, dim), lambda i: (0, 0)),
            pl.BlockSpec((1, dim), lambda i: (0, 0)),
            pl.BlockSpec((1, dim), lambda i: (0, 0)),
            pl.BlockSpec((1, dim), lambda i: (0, 0)),
            pl.BlockSpec((num_codes, dim), lambda i: (0, 0)),
            pl.BlockSpec((1, num_codes), lambda i: (0, 0)),
            pl.BlockSpec((TM, 1), lambda i: (i, 0)),
        ],
        out_specs=pl.BlockSpec((TM,), lambda i: (i,)),
        out_shape=jax.ShapeDtypeStruct((n_tokens,), jnp.int32),
    )(x, mean, denom, gamma2, beta2, codebook, b2, a2)


@functools.lru_cache(maxsize=None)
def _make_sc_gather(num_codes, dim, n_tokens):
    b_per_w = n_tokens // _SC_WORKERS
    mesh = plsc.VectorSubcoreMesh(core_axis_name="c", subcore_axis_name="s")

    @functools.partial(
        pl.kernel, mesh=mesh,
        out_type=jax.ShapeDtypeStruct((n_tokens, dim), jnp.float32),
        scratch_types=[
            pltpu.VMEM((b_per_w,), jnp.int32),
            pltpu.VMEM((b_per_w, dim), jnp.float32),
            pltpu.SemaphoreType.DMA,
        ],
    )
    def gather(table_hbm, idx_hbm, out_hbm, idx_v, rows_v, sem):
        wid = lax.axis_index("s") * _SC_CORES + lax.axis_index("c")
        base = wid * b_per_w
        pltpu.sync_copy(idx_hbm.at[pl.ds(base, b_per_w)], idx_v)
        pltpu.async_copy(table_hbm.at[idx_v], rows_v, sem).wait()
        pltpu.sync_copy(rows_v, out_hbm.at[pl.ds(base, b_per_w)])

    return gather


def kernel(x, bn_gamma, bn_beta, codebook):
    n_tokens, dim = x.shape
    num_codes = codebook.shape[0]
    mean = jnp.mean(x, axis=0, keepdims=True)
    var = jnp.mean((x - mean) ** 2, axis=0, keepdims=True)
    denom = jnp.sqrt(var + EPS)
    b2 = jnp.sum(codebook * codebook, axis=-1)[None, :]
    xn_stat = (x - mean) / denom * bn_gamma + bn_beta
    a2 = jnp.sum(xn_stat * xn_stat, axis=-1, keepdims=True)
    indices = _nearest_indices(
        (x, mean, denom, bn_gamma[None, :], bn_beta[None, :],
         codebook + codebook, b2, a2),
        n_tokens, dim, num_codes)
    codes = _make_sc_gather(num_codes, dim, n_tokens)(codebook, indices)
    return codes, indices.reshape(n_tokens, 1)


# speedup vs baseline: 1.9795x; 1.1237x over previous
import functools

import jax
import jax.numpy as jnp
from jax import lax
from jax.experimental import pallas as pl
from jax.experimental.pallas import tpu as pltpu
from jax.experimental.pallas import tpu_sc as plsc

EPS = 1e-5
TM = 1024
_GCH = 4

_SC_CORES = 2
_SC_SUBCORES = 16
_SC_WORKERS = _SC_CORES * _SC_SUBCORES


def _dist_argmin_body(x_ref, mean_ref, denom_ref, gamma_ref, beta_ref,
                      cb_ref, b2_ref, a2_ref, idx_ref):
    xn = (x_ref[...] - mean_ref[...]) / denom_ref[...] * gamma_ref[...] + beta_ref[...]
    cb2 = cb_ref[...] + cb_ref[...]
    s2 = lax.dot_general(xn, cb2, (((1,), (1,)), ((), ())),
                         preferred_element_type=jnp.float32)
    d2 = a2_ref[...] + b2_ref[...] - s2
    dist = jnp.sqrt(d2)
    idx_ref[...] = jnp.argmin(dist, axis=1).astype(jnp.int32)


def _nearest_indices(xn_inputs, n_rows, dim, num_codes):
    x, mean, denom, gamma2, beta2, codebook, b2, a2 = xn_inputs
    return pl.pallas_call(
        _dist_argmin_body,
        grid=(n_rows // TM,),
        in_specs=[
            pl.BlockSpec((TM, dim), lambda i: (i, 0)),
            pl.BlockSpec((1, dim), lambda i: (0, 0)),
            pl.BlockSpec((1, dim), lambda i: (0, 0)),
            pl.BlockSpec((1, dim), lambda i: (0, 0)),
            pl.BlockSpec((1, dim), lambda i: (0, 0)),
            pl.BlockSpec((num_codes, dim), lambda i: (0, 0)),
            pl.BlockSpec((1, num_codes), lambda i: (0, 0)),
            pl.BlockSpec((TM, 1), lambda i: (i, 0)),
        ],
        out_specs=pl.BlockSpec((TM,), lambda i: (i,)),
        out_shape=jax.ShapeDtypeStruct((n_rows,), jnp.int32),
    )(x, mean, denom, gamma2, beta2, codebook, b2, a2)


@functools.lru_cache(maxsize=None)
def _make_sc_gather(num_codes, dim, n_tokens):
    b_per_w = n_tokens // _SC_WORKERS
    ch = b_per_w // _GCH
    mesh = plsc.VectorSubcoreMesh(core_axis_name="c", subcore_axis_name="s")

    @functools.partial(
        pl.kernel, mesh=mesh,
        out_type=jax.ShapeDtypeStruct((n_tokens, dim), jnp.float32),
        scratch_types=(
            [pltpu.VMEM((b_per_w,), jnp.int32)]
            + [pltpu.VMEM((ch, dim), jnp.float32) for _ in range(_GCH)]
            + [pltpu.SemaphoreType.DMA for _ in range(_GCH)]
            + [pltpu.SemaphoreType.DMA]
        ),
    )
    def gather(table_hbm, idx_hbm, out_hbm, idx_v, *rest):
        rows = rest[:_GCH]
        gsems = rest[_GCH:2 * _GCH]
        wsem = rest[2 * _GCH]
        wid = lax.axis_index("s") * _SC_CORES + lax.axis_index("c")
        base = wid * b_per_w
        pltpu.sync_copy(idx_hbm.at[pl.ds(base, b_per_w)], idx_v)
        gets = [
            pltpu.async_copy(table_hbm.at[idx_v.at[pl.ds(c * ch, ch)]],
                             rows[c], gsems[c])
            for c in range(_GCH)
        ]
        puts = []
        for c in range(_GCH):
            gets[c].wait()
            puts.append(pltpu.async_copy(
                rows[c], out_hbm.at[pl.ds(base + c * ch, ch)], wsem))
        for p in puts:
            p.wait()

    return gather


def kernel(x, bn_gamma, bn_beta, codebook):
    n_tokens, dim = x.shape
    num_codes = codebook.shape[0]
    mean = jnp.mean(x, axis=0, keepdims=True)
    var = jnp.mean((x - mean) ** 2, axis=0, keepdims=True)
    denom = jnp.sqrt(var + EPS)
    b2 = jnp.sum(codebook * codebook, axis=-1)[None, :]
    xn_stat = (x - mean) / denom * bn_gamma + bn_beta
    a2 = jnp.sum(xn_stat * xn_stat, axis=-1, keepdims=True)
    args = (x, mean, denom, bn_gamma[None, :], bn_beta[None, :],
            codebook, b2, a2)
    indices = _nearest_indices(args, n_tokens, dim, num_codes)
    codes = _make_sc_gather(num_codes, dim, n_tokens)(codebook, indices)
    return codes, indices.reshape(n_tokens, 1)


# speedup vs baseline: 2.0496x; 1.0354x over previous
import functools

import jax
import jax.numpy as jnp
from jax import lax
from jax.experimental import pallas as pl
from jax.experimental.pallas import tpu as pltpu
from jax.experimental.pallas import tpu_sc as plsc

EPS = 1e-5
TM = 1024
_GCH = 4

_SC_CORES = 2
_SC_SUBCORES = 16
_SC_WORKERS = _SC_CORES * _SC_SUBCORES


def _dist_argmin_body(x_ref, mean_ref, denom_ref, gamma_ref, beta_ref,
                      cb_ref, b2_ref, a2_ref, idx_ref):
    xn = (x_ref[...] - mean_ref[...]) / denom_ref[...] * gamma_ref[...] + beta_ref[...]
    cb2 = cb_ref[...] + cb_ref[...]
    s2 = lax.dot_general(xn, cb2, (((1,), (1,)), ((), ())),
                         preferred_element_type=jnp.float32)
    d2 = a2_ref[...].reshape(s2.shape[0], 1) + b2_ref[...] - s2
    dist = jnp.sqrt(d2)
    idx_ref[...] = jnp.argmin(dist, axis=1).astype(jnp.int32)


def _nearest_indices(xn_inputs, n_rows, dim, num_codes):
    x, mean, denom, gamma2, beta2, codebook, b2, a2 = xn_inputs
    return pl.pallas_call(
        _dist_argmin_body,
        grid=(n_rows // TM,),
        in_specs=[
            pl.BlockSpec((TM, dim), lambda i: (i, 0)),
            pl.BlockSpec((1, dim), lambda i: (0, 0)),
            pl.BlockSpec((1, dim), lambda i: (0, 0)),
            pl.BlockSpec((1, dim), lambda i: (0, 0)),
            pl.BlockSpec((1, dim), lambda i: (0, 0)),
            pl.BlockSpec((num_codes, dim), lambda i: (0, 0)),
            pl.BlockSpec((1, num_codes), lambda i: (0, 0)),
            pl.BlockSpec((TM,), lambda i: (i,)),
        ],
        out_specs=pl.BlockSpec((TM,), lambda i: (i,)),
        out_shape=jax.ShapeDtypeStruct((n_rows,), jnp.int32),
    )(x, mean, denom, gamma2, beta2, codebook, b2, a2)


@functools.lru_cache(maxsize=None)
def _make_sc_gather(num_codes, dim, n_tokens):
    b_per_w = n_tokens // _SC_WORKERS
    ch = b_per_w // _GCH
    mesh = plsc.VectorSubcoreMesh(core_axis_name="c", subcore_axis_name="s")

    @functools.partial(
        pl.kernel, mesh=mesh,
        out_type=jax.ShapeDtypeStruct((n_tokens, dim), jnp.float32),
        scratch_types=(
            [pltpu.VMEM((b_per_w,), jnp.int32)]
            + [pltpu.VMEM((ch, dim), jnp.float32) for _ in range(_GCH)]
            + [pltpu.SemaphoreType.DMA for _ in range(_GCH)]
            + [pltpu.SemaphoreType.DMA]
        ),
    )
    def gather(table_hbm, idx_hbm, out_hbm, idx_v, *rest):
        rows = rest[:_GCH]
        gsems = rest[_GCH:2 * _GCH]
        wsem = rest[2 * _GCH]
        wid = lax.axis_index("s") * _SC_CORES + lax.axis_index("c")
        base = wid * b_per_w
        pltpu.sync_copy(idx_hbm.at[pl.ds(base, b_per_w)], idx_v)
        gets = [
            pltpu.async_copy(table_hbm.at[idx_v.at[pl.ds(c * ch, ch)]],
                             rows[c], gsems[c])
            for c in range(_GCH)
        ]
        puts = []
        for c in range(_GCH):
            gets[c].wait()
            puts.append(pltpu.async_copy(
                rows[c], out_hbm.at[pl.ds(base + c * ch, ch)], wsem))
        for p in puts:
            p.wait()

    return gather


def kernel(x, bn_gamma, bn_beta, codebook):
    n_tokens, dim = x.shape
    num_codes = codebook.shape[0]
    mean = jnp.mean(x, axis=0, keepdims=True)
    var = jnp.mean((x - mean) ** 2, axis=0, keepdims=True)
    denom = jnp.sqrt(var + EPS)
    b2 = jnp.sum(codebook * codebook, axis=-1)[None, :]
    xn_stat = (x - mean) / denom * bn_gamma + bn_beta
    a2 = jnp.sum(xn_stat * xn_stat, axis=-1)
    args = (x, mean, denom, bn_gamma[None, :], bn_beta[None, :],
            codebook, b2, a2)
    indices = _nearest_indices(args, n_tokens, dim, num_codes)
    codes = _make_sc_gather(num_codes, dim, n_tokens)(codebook, indices)
    return codes, indices.reshape(n_tokens, 1)
